# baseline (speedup 1.0000x reference)
import functools

import jax
import jax.numpy as jnp
from jax import lax
from jax.experimental import pallas as pl
from jax.experimental.pallas import tpu as pltpu
from jax.experimental.pallas import tpu_sc as plsc

NUM_ENTITIES = 1000000
NUM_RELATIONS = 1000
DIM = 64
BATCH = 16384

NC = 2
NS = 16
NW = NC * NS
B_PER_W = BATCH // NW
CHUNK = 128
NCHUNK = B_PER_W // CHUNK
NBLK = B_PER_W // 16


def _sc_body(h_idx_hbm, r_idx_hbm, t_idx_hbm, ent_hbm, rel_hbm, out_hbm,
             hidx_v, ridx_v, tidx_v, hr_v, t_v, m_v, out_v, sem_a, sem_b):
    wid = lax.axis_index("s") * NC + lax.axis_index("c")
    base = wid * B_PER_W

    pltpu.sync_copy(h_idx_hbm.at[pl.ds(base, B_PER_W)], hidx_v)
    pltpu.sync_copy(r_idx_hbm.at[pl.ds(base, B_PER_W)], ridx_v)
    pltpu.sync_copy(t_idx_hbm.at[pl.ds(base, B_PER_W)], tidx_v)

    r_copies = []
    t_copies = []
    for j in range(NCHUNK):
        sl = pl.ds(j * CHUNK, CHUNK)
        r_copies.append(
            pltpu.async_copy(rel_hbm.at[ridx_v.at[sl]], hr_v.at[sl], sem_a))
        t_copies.append(
            pltpu.async_copy(ent_hbm.at[tidx_v.at[sl]], t_v.at[sl], sem_b))
    for c in r_copies:
        c.wait()
    h_copies = []
    for j in range(NCHUNK):
        sl = pl.ds(j * CHUNK, CHUNK)
        h_copies.append(
            pltpu.async_copy(ent_hbm.at[hidx_v.at[sl]], hr_v.at[sl], sem_a,
                             add=True))
    for c in h_copies:
        c.wait()
    for c in t_copies:
        c.wait()

    lanes = lax.iota(jnp.int32, 16)

    def _sqrt16(x):
        xs = jnp.maximum(x, jnp.float32(1e-30))
        i = plsc.bitcast(xs, jnp.int32)
        i = jnp.int32(0x5F3759DF) - lax.shift_right_arithmetic(i, jnp.int32(1))
        y = plsc.bitcast(i, jnp.float32)
        half = jnp.float32(0.5) * xs
        for _ in range(3):
            y = y * (jnp.float32(1.5) - half * y * y)
        return xs * y

    def block_body(i, carry):
        b0 = i * 16
        for row in range(16):
            b = b0 + row
            acc = jnp.zeros((16,), jnp.float32)
            for s in range(DIM // 16):
                d = hr_v[b, pl.ds(s * 16, 16)] - t_v[b, pl.ds(s * 16, 16)]
                acc = acc + d * d
            m_v[row, pl.ds(0, 16)] = acc
        tot = jnp.zeros((16,), jnp.float32)
        for j in range(16):
            col = plsc.load_gather(
                m_v, [lanes, jnp.full((16,), j, jnp.int32)])
            tot = tot + col
        out_v[pl.ds(b0, 16)] = _sqrt16(tot)
        return carry

    lax.fori_loop(0, NBLK, block_body, 0)

    pltpu.sync_copy(out_v, out_hbm.at[pl.ds(base, B_PER_W)])


@jax.jit
def _transe_sc(h_idx, r_idx, t_idx, entity_emb, rel_emb):
    mesh = plsc.VectorSubcoreMesh(core_axis_name="c", subcore_axis_name="s")
    return pl.kernel(
        _sc_body,
        out_type=jax.ShapeDtypeStruct((BATCH,), jnp.float32),
        mesh=mesh,
        compiler_params=pltpu.CompilerParams(
            needs_layout_passes=False, use_tc_tiling_on_sc=False),
        scratch_types=[
            pltpu.VMEM((B_PER_W,), jnp.int32),
            pltpu.VMEM((B_PER_W,), jnp.int32),
            pltpu.VMEM((B_PER_W,), jnp.int32),
            pltpu.VMEM((B_PER_W, DIM), jnp.float32),
            pltpu.VMEM((B_PER_W, DIM), jnp.float32),
            pltpu.VMEM((16, 17), jnp.float32),
            pltpu.VMEM((B_PER_W,), jnp.float32),
            pltpu.SemaphoreType.DMA,
            pltpu.SemaphoreType.DMA,
        ],
    )(h_idx, r_idx, t_idx, entity_emb, rel_emb)


def kernel(h_idx, r_idx, t_idx, entity_emb, rel_emb):
    return _transe_sc(h_idx.astype(jnp.int32), r_idx.astype(jnp.int32),
                      t_idx.astype(jnp.int32), entity_emb, rel_emb)
